# baseline (speedup 1.0000x reference)
import functools

import jax
import jax.numpy as jnp
from jax import lax
from jax.experimental import pallas as pl
from jax.experimental.pallas import tpu as pltpu
from jax.experimental.pallas import tpu_sc as plsc

BS, C, NY, NX = 8, 192, 64, 64
NC, NA = 80, 1
K = 8192
NPIX = BS * NY * NX
BLK = 1024
STEM_ROWS = NPIX + BLK

NWORK = 32
BPW = K // NWORK
GCH = 128
NCH = BPW // GCH

KB = 512
OUT_W = 4 + 1 + NC
CP = 256
CH = 128


def _stem_body(x_ref, w_ref, b_ref, o_ref):
    i = pl.program_id(0)
    last = pl.num_programs(0) - 1

    @pl.when(i == last)
    def _():
        o_ref[...] = jnp.zeros_like(o_ref)

    @pl.when(i < last)
    def _():
        acc = lax.dot_general(x_ref[...], w_ref[...], (((1,), (0,)), ((), ())),
                              preferred_element_type=jnp.float32)
        acc = acc + b_ref[...]
        act = acc * jax.nn.sigmoid(acc)
        lo = lax.bitcast_convert_type(
            act[:, :CH].astype(jnp.bfloat16), jnp.uint16).astype(jnp.uint32)
        hi = lax.bitcast_convert_type(
            act[:, CH:].astype(jnp.bfloat16), jnp.uint16).astype(jnp.uint32)
        o_ref[...] = lax.bitcast_convert_type((hi << 16) | lo, jnp.int32)


def _stem(xt, stem_wt, stem_b2):
    nblk = STEM_ROWS // BLK
    cap = NPIX // BLK - 1
    return pl.pallas_call(
        _stem_body,
        grid=(nblk,),
        in_specs=[
            pl.BlockSpec((BLK, C), lambda i: (jnp.minimum(i, cap), 0)),
            pl.BlockSpec((C, CP), lambda i: (0, 0)),
            pl.BlockSpec((1, CP), lambda i: (0, 0)),
        ],
        out_specs=pl.BlockSpec((BLK, CH), lambda i: (i, 0)),
        out_shape=jax.ShapeDtypeStruct((STEM_ROWS, CH), jnp.int32),
    )(xt, stem_wt, stem_b2)


_TAPS = [(dy, dx) for dy in (-1, 0, 1) for dx in (-1, 0, 1)]


DEPTH = 4


def _gather_body(stem_hbm, bi_hbm, yi_hbm, xi_hbm, g_hbm, *refs):
    (bv, yv, xv), taps = refs[0:3], refs[3:12]
    bufs = refs[12:12 + DEPTH]
    gsem = refs[12 + DEPTH:12 + 2 * DEPTH]
    wsem = refs[12 + 2 * DEPTH:12 + 3 * DEPTH]
    wid = lax.axis_index("s") * 2 + lax.axis_index("c")
    base = wid * BPW
    pltpu.sync_copy(bi_hbm.at[pl.ds(base, BPW)], bv)
    pltpu.sync_copy(yi_hbm.at[pl.ds(base, BPW)], yv)
    pltpu.sync_copy(xi_hbm.at[pl.ds(base, BPW)], xv)

    for j in range(BPW // 16):
        sl = pl.ds(j * 16, 16)
        b = bv[sl]
        y = yv[sl]
        x = xv[sl]
        flat = (b * NY + y) * NX + x
        y_lo = y >= 1
        y_hi = y <= NY - 2
        x_lo = x >= 1
        x_hi = x <= NX - 2
        for t, (dy, dx) in enumerate(_TAPS):
            ok = None
            if dy == -1:
                ok = y_lo
            elif dy == 1:
                ok = y_hi
            if dx == -1:
                ok = x_lo if ok is None else (ok & x_lo)
            elif dx == 1:
                ok = x_hi if ok is None else (ok & x_hi)
            ft = flat + (dy * NX + dx)
            if ok is not None:
                ft = jnp.where(ok, ft, NPIX)
            taps[t][sl] = ft

    chunks = [(t, cc) for t in range(9) for cc in range(NCH)]
    n = len(chunks)

    def _start_gather(i, b):
        t, cc = chunks[i]
        return pltpu.async_copy(
            stem_hbm.at[taps[t].at[pl.ds(cc * GCH, GCH)]], bufs[b], gsem[b])

    def _start_write(i, b):
        t, cc = chunks[i]
        return pltpu.async_copy(
            bufs[b], g_hbm.at[t, pl.ds(base + cc * GCH, GCH)], wsem[b])

    gdesc = [None] * DEPTH
    wdesc = [None] * DEPTH
    for i in range(min(DEPTH, n)):
        gdesc[i] = _start_gather(i, i)
    for i in range(n):
        b = i % DEPTH
        gdesc[b].wait()
        wdesc[b] = _start_write(i, b)
        j = i + DEPTH
        if j < n:
            wdesc[b].wait()
            gdesc[b] = _start_gather(j, b)
            wdesc[b] = None
    for b in range(DEPTH):
        if wdesc[b] is not None:
            wdesc[b].wait()


def _gather(stem_full, bi, yi, xi):
    mesh = plsc.VectorSubcoreMesh(core_axis_name="c", subcore_axis_name="s")
    return pl.kernel(
        _gather_body,
        out_type=jax.ShapeDtypeStruct((9, K, CH), jnp.int32),
        mesh=mesh,
        scratch_types=(
            [pltpu.VMEM((BPW,), jnp.int32) for _ in range(3)]
            + [pltpu.VMEM((BPW,), jnp.int32) for _ in range(9)]
            + [pltpu.VMEM((GCH, CH), jnp.int32) for _ in range(DEPTH)]
            + [pltpu.SemaphoreType.DMA for _ in range(3 * DEPTH)]
        ),
    )(stem_full, bi, yi, xi)


def _head_body(g_ref, wc_ref, wr_ref, cb_ref, rb_ref, wro_ref, wcb_ref,
               hb_ref, o_ref):
    accc = jnp.zeros((KB, C), jnp.float32) + cb_ref[...]
    accr = jnp.zeros((KB, C), jnp.float32) + rb_ref[...]
    for t in range(9):
        gu = lax.bitcast_convert_type(g_ref[t], jnp.uint32)
        glo = lax.bitcast_convert_type(gu << 16, jnp.float32)
        ghi = lax.bitcast_convert_type(gu & jnp.uint32(0xFFFF0000), jnp.float32)
        accc += lax.dot_general(glo, wc_ref[t][:, :CH], (((1,), (1,)), ((), ())),
                                preferred_element_type=jnp.float32)
        accc += lax.dot_general(ghi, wc_ref[t][:, CH:], (((1,), (1,)), ((), ())),
                                preferred_element_type=jnp.float32)
        accr += lax.dot_general(glo, wr_ref[t][:, :CH], (((1,), (1,)), ((), ())),
                                preferred_element_type=jnp.float32)
        accr += lax.dot_general(ghi, wr_ref[t][:, CH:], (((1,), (1,)), ((), ())),
                                preferred_element_type=jnp.float32)
    cls_f = accc * jax.nn.sigmoid(accc)
    reg_f = accr * jax.nn.sigmoid(accr)
    out = lax.dot_general(reg_f, wro_ref[...], (((1,), (1,)), ((), ())),
                          preferred_element_type=jnp.float32)
    out += lax.dot_general(cls_f, wcb_ref[...], (((1,), (1,)), ((), ())),
                           preferred_element_type=jnp.float32)
    o_ref[...] = out + hb_ref[...]


def _heads(g, w9c, w9r, cb2, rb2, wro, wcb, hbias):
    return pl.pallas_call(
        _head_body,
        grid=(K // KB,),
        in_specs=[
            pl.BlockSpec((9, KB, CH), lambda i: (0, i, 0)),
            pl.BlockSpec((9, C, CP), lambda i: (0, 0, 0)),
            pl.BlockSpec((9, C, CP), lambda i: (0, 0, 0)),
            pl.BlockSpec((1, C), lambda i: (0, 0)),
            pl.BlockSpec((1, C), lambda i: (0, 0)),
            pl.BlockSpec((OUT_W, C), lambda i: (0, 0)),
            pl.BlockSpec((OUT_W, C), lambda i: (0, 0)),
            pl.BlockSpec((1, OUT_W), lambda i: (0, 0)),
        ],
        out_specs=pl.BlockSpec((KB, OUT_W), lambda i: (i, 0)),
        out_shape=jax.ShapeDtypeStruct((K, OUT_W), jnp.float32),
    )(g, w9c, w9r, cb2, rb2, wro, wcb, hbias)


def kernel(x, indices, stem_w, stem_b, cls_conv_w, cls_conv_b,
           reg_conv_w, reg_conv_b, cls_pred_w, cls_pred_b,
           reg_pred_w, reg_pred_b, obj_pred_w, obj_pred_b):
    xt = jnp.transpose(x.reshape(BS, C, NY * NX), (0, 2, 1)).reshape(NPIX, C)
    stem_wt = jnp.pad(stem_w, ((0, CP - C), (0, 0))).T
    stem_bp = jnp.pad(stem_b, (0, CP - C)).reshape(1, CP)
    stem_full = _stem(xt, stem_wt, stem_bp)

    idx32 = indices.astype(jnp.int32)
    g = _gather(stem_full, idx32[:, 0], idx32[:, 1], idx32[:, 2])

    def _tapw(w):
        return jnp.pad(jnp.transpose(w.reshape(C, C, 9), (2, 0, 1)),
                       ((0, 0), (0, 0), (0, CP - C)))

    w9c = _tapw(cls_conv_w)
    w9r = _tapw(reg_conv_w)
    wro = jnp.concatenate(
        [reg_pred_w, obj_pred_w, jnp.zeros((NC, C), jnp.float32)], axis=0)
    wcb = jnp.concatenate(
        [jnp.zeros((4 + 1, C), jnp.float32), cls_pred_w], axis=0)
    hbias = jnp.concatenate(
        [reg_pred_b, obj_pred_b, cls_pred_b]).reshape(1, OUT_W)

    return _heads(g, w9c, w9r, cls_conv_b.reshape(1, C),
                  reg_conv_b.reshape(1, C), wro, wcb, hbias)


# speedup vs baseline: 1.2611x; 1.2611x over previous
import functools

import jax
import jax.numpy as jnp
from jax import lax
from jax.experimental import pallas as pl
from jax.experimental.pallas import tpu as pltpu
from jax.experimental.pallas import tpu_sc as plsc

BS, C, NY, NX = 8, 192, 64, 64
NC, NA = 80, 1
K = 8192
NPIX = BS * NY * NX
BLK = 1024
STEM_ROWS = NPIX + BLK

NWORK = 32
BPW = K // NWORK
GCH = 128
NCH = BPW // GCH

KB = 512
OUT_W = 4 + 1 + NC
CP = 256
CH = 128


def _stem_body(x_ref, w_ref, b_ref, o_ref):
    i = pl.program_id(0)
    last = pl.num_programs(0) - 1

    @pl.when(i == last)
    def _():
        o_ref[...] = jnp.zeros_like(o_ref)

    @pl.when(i < last)
    def _():
        acc = lax.dot_general(x_ref[...], w_ref[...], (((1,), (0,)), ((), ())),
                              preferred_element_type=jnp.float32)
        acc = acc + b_ref[...]
        act = acc * jax.nn.sigmoid(acc)
        lo = lax.bitcast_convert_type(
            act[:, :CH].astype(jnp.bfloat16), jnp.uint16).astype(jnp.uint32)
        hi = lax.bitcast_convert_type(
            act[:, CH:].astype(jnp.bfloat16), jnp.uint16).astype(jnp.uint32)
        o_ref[...] = lax.bitcast_convert_type((hi << 16) | lo, jnp.int32)


def _stem(xt, stem_wt, stem_b2):
    nblk = STEM_ROWS // BLK
    cap = NPIX // BLK - 1
    return pl.pallas_call(
        _stem_body,
        grid=(nblk,),
        in_specs=[
            pl.BlockSpec((BLK, C), lambda i: (jnp.minimum(i, cap), 0)),
            pl.BlockSpec((C, CP), lambda i: (0, 0)),
            pl.BlockSpec((1, CP), lambda i: (0, 0)),
        ],
        out_specs=pl.BlockSpec((BLK, CH), lambda i: (i, 0)),
        out_shape=jax.ShapeDtypeStruct((STEM_ROWS, CH), jnp.int32),
    )(xt, stem_wt, stem_b2)


_TAPS = [(dy, dx) for dy in (-1, 0, 1) for dx in (-1, 0, 1)]


DEPTH = 2


def _gather_body(stem_hbm, bi_hbm, yi_hbm, xi_hbm, g_hbm, *refs):
    (bv, yv, xv), taps = refs[0:3], refs[3:6]
    bufs = refs[6:6 + DEPTH]
    gsem = refs[6 + DEPTH:6 + 2 * DEPTH]
    wsem = refs[6 + 2 * DEPTH:6 + 3 * DEPTH]
    wid = lax.axis_index("s") * 2 + lax.axis_index("c")
    base = wid * BPW
    pltpu.sync_copy(bi_hbm.at[pl.ds(base, BPW)], bv)
    pltpu.sync_copy(yi_hbm.at[pl.ds(base, BPW)], yv)
    pltpu.sync_copy(xi_hbm.at[pl.ds(base, BPW)], xv)

    for j in range(BPW // 16):
        sl = pl.ds(j * 16, 16)
        b = bv[sl]
        y = yv[sl]
        x = xv[sl]
        flat = (b * NY + y) * NX + x
        taps[0][sl] = jnp.where(y >= 1, flat - NX, NPIX)
        taps[1][sl] = flat
        taps[2][sl] = jnp.where(y <= NY - 2, flat + NX, NPIX)

    chunks = [(t, cc) for t in range(3) for cc in range(NCH)]
    n = len(chunks)

    def _start_gather(i, b):
        t, cc = chunks[i]
        return pltpu.async_copy(
            stem_hbm.at[taps[t].at[pl.ds(cc * GCH, GCH)]], bufs[b], gsem[b])

    def _start_write(i, b):
        t, cc = chunks[i]
        return pltpu.async_copy(
            bufs[b], g_hbm.at[t, pl.ds(base + cc * GCH, GCH)], wsem[b])

    gdesc = [None] * DEPTH
    wdesc = [None] * DEPTH
    for i in range(min(DEPTH, n)):
        gdesc[i] = _start_gather(i, i)
    for i in range(n):
        b = i % DEPTH
        gdesc[b].wait()
        wdesc[b] = _start_write(i, b)
        j = i + DEPTH
        if j < n:
            wdesc[b].wait()
            gdesc[b] = _start_gather(j, b)
            wdesc[b] = None
    for b in range(DEPTH):
        if wdesc[b] is not None:
            wdesc[b].wait()


def _gather(stem_full, bi, yi, xi):
    mesh = plsc.VectorSubcoreMesh(core_axis_name="c", subcore_axis_name="s")
    return pl.kernel(
        _gather_body,
        out_type=jax.ShapeDtypeStruct((3, K, 3 * CH), jnp.int32),
        mesh=mesh,
        scratch_types=(
            [pltpu.VMEM((BPW,), jnp.int32) for _ in range(3)]
            + [pltpu.VMEM((BPW,), jnp.int32) for _ in range(3)]
            + [pltpu.VMEM((GCH, 3 * CH), jnp.int32) for _ in range(DEPTH)]
            + [pltpu.SemaphoreType.DMA for _ in range(3 * DEPTH)]
        ),
    )(stem_full, bi, yi, xi)


def _shift3_body(p_ref, c_ref, n_ref, o_ref):
    cur = c_ref[...]
    pm1 = jnp.concatenate([p_ref[BLK - 1:, :], cur[:BLK - 1, :]], axis=0)
    pp1 = jnp.concatenate([cur[1:, :], n_ref[:1, :]], axis=0)
    xloc = lax.broadcasted_iota(jnp.int32, (BLK, CH), 0) % NX
    zero = jnp.zeros((BLK, CH), jnp.int32)
    o_ref[:, 0:CH] = jnp.where(xloc != 0, pm1, zero)
    o_ref[:, CH:2 * CH] = cur
    o_ref[:, 2 * CH:] = jnp.where(xloc != NX - 1, pp1, zero)


def _shift3(stem_pk):
    nblk = STEM_ROWS // BLK
    return pl.pallas_call(
        _shift3_body,
        grid=(nblk,),
        in_specs=[
            pl.BlockSpec((BLK, CH), lambda i: (jnp.maximum(i - 1, 0), 0)),
            pl.BlockSpec((BLK, CH), lambda i: (i, 0)),
            pl.BlockSpec((BLK, CH), lambda i: (jnp.minimum(i + 1, STEM_ROWS // BLK - 1), 0)),
        ],
        out_specs=pl.BlockSpec((BLK, 3 * CH), lambda i: (i, 0)),
        out_shape=jax.ShapeDtypeStruct((STEM_ROWS, 3 * CH), jnp.int32),
    )(stem_pk, stem_pk, stem_pk)


def _head_body(g_ref, wcl_ref, wch_ref, wrl_ref, wrh_ref, cb_ref, rb_ref,
               wro_ref, wcb_ref, hb_ref, o_ref):
    accc = jnp.zeros((KB, C), jnp.float32) + cb_ref[...]
    accr = jnp.zeros((KB, C), jnp.float32) + rb_ref[...]
    for d in range(3):
        gu = lax.bitcast_convert_type(g_ref[d], jnp.uint32)
        glo = lax.bitcast_convert_type(gu << 16, jnp.float32)
        ghi = lax.bitcast_convert_type(gu & jnp.uint32(0xFFFF0000), jnp.float32)
        accc += lax.dot_general(glo, wcl_ref[d], (((1,), (1,)), ((), ())),
                                preferred_element_type=jnp.float32)
        accc += lax.dot_general(ghi, wch_ref[d], (((1,), (1,)), ((), ())),
                                preferred_element_type=jnp.float32)
        accr += lax.dot_general(glo, wrl_ref[d], (((1,), (1,)), ((), ())),
                                preferred_element_type=jnp.float32)
        accr += lax.dot_general(ghi, wrh_ref[d], (((1,), (1,)), ((), ())),
                                preferred_element_type=jnp.float32)
    cls_f = accc * jax.nn.sigmoid(accc)
    reg_f = accr * jax.nn.sigmoid(accr)
    out = lax.dot_general(reg_f, wro_ref[...], (((1,), (1,)), ((), ())),
                          preferred_element_type=jnp.float32)
    out += lax.dot_general(cls_f, wcb_ref[...], (((1,), (1,)), ((), ())),
                           preferred_element_type=jnp.float32)
    o_ref[...] = out + hb_ref[...]


def _heads(g, wcl, wch, wrl, wrh, cb2, rb2, wro, wcb, hbias):
    return pl.pallas_call(
        _head_body,
        grid=(K // KB,),
        in_specs=[
            pl.BlockSpec((3, KB, 3 * CH), lambda i: (0, i, 0)),
            pl.BlockSpec((3, C, 3 * CH), lambda i: (0, 0, 0)),
            pl.BlockSpec((3, C, 3 * CH), lambda i: (0, 0, 0)),
            pl.BlockSpec((3, C, 3 * CH), lambda i: (0, 0, 0)),
            pl.BlockSpec((3, C, 3 * CH), lambda i: (0, 0, 0)),
            pl.BlockSpec((1, C), lambda i: (0, 0)),
            pl.BlockSpec((1, C), lambda i: (0, 0)),
            pl.BlockSpec((OUT_W, C), lambda i: (0, 0)),
            pl.BlockSpec((OUT_W, C), lambda i: (0, 0)),
            pl.BlockSpec((1, OUT_W), lambda i: (0, 0)),
        ],
        out_specs=pl.BlockSpec((KB, OUT_W), lambda i: (i, 0)),
        out_shape=jax.ShapeDtypeStruct((K, OUT_W), jnp.float32),
    )(g, wcl, wch, wrl, wrh, cb2, rb2, wro, wcb, hbias)


def kernel(x, indices, stem_w, stem_b, cls_conv_w, cls_conv_b,
           reg_conv_w, reg_conv_b, cls_pred_w, cls_pred_b,
           reg_pred_w, reg_pred_b, obj_pred_w, obj_pred_b):
    xt = jnp.transpose(x.reshape(BS, C, NY * NX), (0, 2, 1)).reshape(NPIX, C)
    stem_wt = jnp.pad(stem_w, ((0, CP - C), (0, 0))).T
    stem_bp = jnp.pad(stem_b, (0, CP - C)).reshape(1, CP)
    stem_full = _stem(xt, stem_wt, stem_bp)

    stem3 = _shift3(stem_full)
    idx32 = indices.astype(jnp.int32)
    g = _gather(stem3, idx32[:, 0], idx32[:, 1], idx32[:, 2])

    def _tapw(w):
        wp = jnp.pad(jnp.transpose(w.reshape(C, C, 9), (2, 0, 1)),
                     ((0, 0), (0, 0), (0, CP - C)))
        lo = wp[:, :, :CH].reshape(3, 3, C, CH).transpose(0, 2, 1, 3).reshape(
            3, C, 3 * CH)
        hi = wp[:, :, CH:].reshape(3, 3, C, CH).transpose(0, 2, 1, 3).reshape(
            3, C, 3 * CH)
        return lo, hi

    wcl, wch = _tapw(cls_conv_w)
    wrl, wrh = _tapw(reg_conv_w)
    wro = jnp.concatenate(
        [reg_pred_w, obj_pred_w, jnp.zeros((NC, C), jnp.float32)], axis=0)
    wcb = jnp.concatenate(
        [jnp.zeros((4 + 1, C), jnp.float32), cls_pred_w], axis=0)
    hbias = jnp.concatenate(
        [reg_pred_b, obj_pred_b, cls_pred_b]).reshape(1, OUT_W)

    return _heads(g, wcl, wch, wrl, wrh, cls_conv_b.reshape(1, C),
                  reg_conv_b.reshape(1, C), wro, wcb, hbias)


# speedup vs baseline: 1.5167x; 1.2027x over previous
import functools

import jax
import jax.numpy as jnp
from jax import lax
from jax.experimental import pallas as pl
from jax.experimental.pallas import tpu as pltpu
from jax.experimental.pallas import tpu_sc as plsc

BS, C, NY, NX = 8, 192, 64, 64
NC, NA = 80, 1
K = 8192
NPIX = BS * NY * NX
BLK = 1024
STEM_ROWS = NPIX + BLK

NWORK = 32
BPW = K // NWORK
GCH = 128
NCH = BPW // GCH

KB = 512
OUT_W = 4 + 1 + NC
CP = 256
CH = 128


def _stem_body(x_ref, w_ref, b_ref, o_ref):
    i = pl.program_id(0)
    last = pl.num_programs(0) - 1

    @pl.when(i == last)
    def _():
        o_ref[...] = jnp.zeros_like(o_ref)

    @pl.when(i < last)
    def _():
        acc = lax.dot_general(x_ref[...], w_ref[...], (((1,), (0,)), ((), ())),
                              preferred_element_type=jnp.float32)
        acc = acc + b_ref[...]
        act = acc * jax.nn.sigmoid(acc)
        lo = lax.bitcast_convert_type(
            act[:, :CH].astype(jnp.bfloat16), jnp.uint16).astype(jnp.uint32)
        hi = lax.bitcast_convert_type(
            act[:, CH:].astype(jnp.bfloat16), jnp.uint16).astype(jnp.uint32)
        pk = lax.bitcast_convert_type((hi << 16) | lo, jnp.int32)
        zrow = jnp.zeros((1, CH), jnp.int32)
        pm1 = jnp.concatenate([zrow, pk[:BLK - 1, :]], axis=0)
        pp1 = jnp.concatenate([pk[1:, :], zrow], axis=0)
        xloc = lax.broadcasted_iota(jnp.int32, (BLK, CH), 0) % NX
        zero = jnp.zeros((BLK, CH), jnp.int32)
        o_ref[:, 0:CH] = jnp.where(xloc != 0, pm1, zero)
        o_ref[:, CH:2 * CH] = pk
        o_ref[:, 2 * CH:] = jnp.where(xloc != NX - 1, pp1, zero)


def _stem(xt, stem_wt, stem_b2):
    nblk = STEM_ROWS // BLK
    cap = NPIX // BLK - 1
    return pl.pallas_call(
        _stem_body,
        grid=(nblk,),
        in_specs=[
            pl.BlockSpec((BLK, C), lambda i: (jnp.minimum(i, cap), 0)),
            pl.BlockSpec((C, CP), lambda i: (0, 0)),
            pl.BlockSpec((1, CP), lambda i: (0, 0)),
        ],
        out_specs=pl.BlockSpec((BLK, 3 * CH), lambda i: (i, 0)),
        out_shape=jax.ShapeDtypeStruct((STEM_ROWS, 3 * CH), jnp.int32),
    )(xt, stem_wt, stem_b2)


_TAPS = [(dy, dx) for dy in (-1, 0, 1) for dx in (-1, 0, 1)]


DEPTH = 2


def _gather_body(stem_hbm, bi_hbm, yi_hbm, xi_hbm, g_hbm, *refs):
    (bv, yv, xv), taps = refs[0:3], refs[3:6]
    bufs = refs[6:6 + DEPTH]
    gsem = refs[6 + DEPTH:6 + 2 * DEPTH]
    wsem = refs[6 + 2 * DEPTH:6 + 3 * DEPTH]
    wid = lax.axis_index("s") * 2 + lax.axis_index("c")
    base = wid * BPW
    pltpu.sync_copy(bi_hbm.at[pl.ds(base, BPW)], bv)
    pltpu.sync_copy(yi_hbm.at[pl.ds(base, BPW)], yv)
    pltpu.sync_copy(xi_hbm.at[pl.ds(base, BPW)], xv)

    for j in range(BPW // 16):
        sl = pl.ds(j * 16, 16)
        b = bv[sl]
        y = yv[sl]
        x = xv[sl]
        flat = (b * NY + y) * NX + x
        taps[0][sl] = jnp.where(y >= 1, flat - NX, NPIX)
        taps[1][sl] = flat
        taps[2][sl] = jnp.where(y <= NY - 2, flat + NX, NPIX)

    chunks = [(t, cc) for t in range(3) for cc in range(NCH)]
    n = len(chunks)

    def _start_gather(i, b):
        t, cc = chunks[i]
        return pltpu.async_copy(
            stem_hbm.at[taps[t].at[pl.ds(cc * GCH, GCH)]], bufs[b], gsem[b])

    def _start_write(i, b):
        t, cc = chunks[i]
        return pltpu.async_copy(
            bufs[b], g_hbm.at[t, pl.ds(base + cc * GCH, GCH)], wsem[b])

    gdesc = [None] * DEPTH
    wdesc = [None] * DEPTH
    for i in range(min(DEPTH, n)):
        gdesc[i] = _start_gather(i, i)
    for i in range(n):
        b = i % DEPTH
        gdesc[b].wait()
        wdesc[b] = _start_write(i, b)
        j = i + DEPTH
        if j < n:
            wdesc[b].wait()
            gdesc[b] = _start_gather(j, b)
            wdesc[b] = None
    for b in range(DEPTH):
        if wdesc[b] is not None:
            wdesc[b].wait()


def _gather(stem_full, bi, yi, xi):
    mesh = plsc.VectorSubcoreMesh(core_axis_name="c", subcore_axis_name="s")
    return pl.kernel(
        _gather_body,
        out_type=jax.ShapeDtypeStruct((3, K, 3 * CH), jnp.int32),
        mesh=mesh,
        scratch_types=(
            [pltpu.VMEM((BPW,), jnp.int32) for _ in range(3)]
            + [pltpu.VMEM((BPW,), jnp.int32) for _ in range(3)]
            + [pltpu.VMEM((GCH, 3 * CH), jnp.int32) for _ in range(DEPTH)]
            + [pltpu.SemaphoreType.DMA for _ in range(3 * DEPTH)]
        ),
    )(stem_full, bi, yi, xi)


def _head_body(g_ref, wcl_ref, wch_ref, wrl_ref, wrh_ref, cb_ref, rb_ref,
               wro_ref, wcb_ref, hb_ref, o_ref):
    accc = jnp.zeros((KB, C), jnp.float32) + cb_ref[...]
    accr = jnp.zeros((KB, C), jnp.float32) + rb_ref[...]
    for d in range(3):
        gu = lax.bitcast_convert_type(g_ref[d], jnp.uint32)
        glo = lax.bitcast_convert_type(gu << 16, jnp.float32)
        ghi = lax.bitcast_convert_type(gu & jnp.uint32(0xFFFF0000), jnp.float32)
        accc += lax.dot_general(glo, wcl_ref[d], (((1,), (1,)), ((), ())),
                                preferred_element_type=jnp.float32)
        accc += lax.dot_general(ghi, wch_ref[d], (((1,), (1,)), ((), ())),
                                preferred_element_type=jnp.float32)
        accr += lax.dot_general(glo, wrl_ref[d], (((1,), (1,)), ((), ())),
                                preferred_element_type=jnp.float32)
        accr += lax.dot_general(ghi, wrh_ref[d], (((1,), (1,)), ((), ())),
                                preferred_element_type=jnp.float32)
    cls_f = accc * jax.nn.sigmoid(accc)
    reg_f = accr * jax.nn.sigmoid(accr)
    out = lax.dot_general(reg_f, wro_ref[...], (((1,), (1,)), ((), ())),
                          preferred_element_type=jnp.float32)
    out += lax.dot_general(cls_f, wcb_ref[...], (((1,), (1,)), ((), ())),
                           preferred_element_type=jnp.float32)
    o_ref[...] = out + hb_ref[...]


def _heads(g, wcl, wch, wrl, wrh, cb2, rb2, wro, wcb, hbias):
    return pl.pallas_call(
        _head_body,
        grid=(K // KB,),
        in_specs=[
            pl.BlockSpec((3, KB, 3 * CH), lambda i: (0, i, 0)),
            pl.BlockSpec((3, C, 3 * CH), lambda i: (0, 0, 0)),
            pl.BlockSpec((3, C, 3 * CH), lambda i: (0, 0, 0)),
            pl.BlockSpec((3, C, 3 * CH), lambda i: (0, 0, 0)),
            pl.BlockSpec((3, C, 3 * CH), lambda i: (0, 0, 0)),
            pl.BlockSpec((1, C), lambda i: (0, 0)),
            pl.BlockSpec((1, C), lambda i: (0, 0)),
            pl.BlockSpec((OUT_W, C), lambda i: (0, 0)),
            pl.BlockSpec((OUT_W, C), lambda i: (0, 0)),
            pl.BlockSpec((1, OUT_W), lambda i: (0, 0)),
        ],
        out_specs=pl.BlockSpec((KB, OUT_W), lambda i: (i, 0)),
        out_shape=jax.ShapeDtypeStruct((K, OUT_W), jnp.float32),
    )(g, wcl, wch, wrl, wrh, cb2, rb2, wro, wcb, hbias)


def kernel(x, indices, stem_w, stem_b, cls_conv_w, cls_conv_b,
           reg_conv_w, reg_conv_b, cls_pred_w, cls_pred_b,
           reg_pred_w, reg_pred_b, obj_pred_w, obj_pred_b):
    xt = jnp.transpose(x.reshape(BS, C, NY * NX), (0, 2, 1)).reshape(NPIX, C)
    stem_wt = jnp.pad(stem_w, ((0, CP - C), (0, 0))).T
    stem_bp = jnp.pad(stem_b, (0, CP - C)).reshape(1, CP)
    stem_full = _stem(xt, stem_wt, stem_bp)

    idx32 = indices.astype(jnp.int32)
    g = _gather(stem_full, idx32[:, 0], idx32[:, 1], idx32[:, 2])

    def _tapw(w):
        wp = jnp.pad(jnp.transpose(w.reshape(C, C, 9), (2, 0, 1)),
                     ((0, 0), (0, 0), (0, CP - C)))
        lo = wp[:, :, :CH].reshape(3, 3, C, CH).transpose(0, 2, 1, 3).reshape(
            3, C, 3 * CH)
        hi = wp[:, :, CH:].reshape(3, 3, C, CH).transpose(0, 2, 1, 3).reshape(
            3, C, 3 * CH)
        return lo, hi

    wcl, wch = _tapw(cls_conv_w)
    wrl, wrh = _tapw(reg_conv_w)
    wro = jnp.concatenate(
        [reg_pred_w, obj_pred_w, jnp.zeros((NC, C), jnp.float32)], axis=0)
    wcb = jnp.concatenate(
        [jnp.zeros((4 + 1, C), jnp.float32), cls_pred_w], axis=0)
    hbias = jnp.concatenate(
        [reg_pred_b, obj_pred_b, cls_pred_b]).reshape(1, OUT_W)

    return _heads(g, wcl, wch, wrl, wrh, cls_conv_b.reshape(1, C),
                  reg_conv_b.reshape(1, C), wro, wcb, hbias)


# speedup vs baseline: 1.5890x; 1.0477x over previous
import functools

import jax
import jax.numpy as jnp
from jax import lax
from jax.experimental import pallas as pl
from jax.experimental.pallas import tpu as pltpu
from jax.experimental.pallas import tpu_sc as plsc

BS, C, NY, NX = 8, 192, 64, 64
NC, NA = 80, 1
K = 8192
NPIX = BS * NY * NX
BLK = 1024
STEM_ROWS = NPIX + BLK

NWORK = 32
SPLIT = 2
KH = K // SPLIT
BPW = KH // NWORK
GCH = 128
NCH = BPW // GCH

KB = 512
OUT_W = 4 + 1 + NC
CP = 256
CH = 128


def _stem_body(x_ref, w_ref, b_ref, o_ref):
    i = pl.program_id(0)
    last = pl.num_programs(0) - 1

    @pl.when(i == last)
    def _():
        o_ref[...] = jnp.zeros_like(o_ref)

    @pl.when(i < last)
    def _():
        acc = lax.dot_general(x_ref[...], w_ref[...], (((1,), (0,)), ((), ())),
                              preferred_element_type=jnp.float32)
        acc = acc + b_ref[...]
        act = acc * jax.nn.sigmoid(acc)
        lo = lax.bitcast_convert_type(
            act[:, :CH].astype(jnp.bfloat16), jnp.uint16).astype(jnp.uint32)
        hi = lax.bitcast_convert_type(
            act[:, CH:].astype(jnp.bfloat16), jnp.uint16).astype(jnp.uint32)
        pk = lax.bitcast_convert_type((hi << 16) | lo, jnp.int32)
        zrow = jnp.zeros((1, CH), jnp.int32)
        pm1 = jnp.concatenate([zrow, pk[:BLK - 1, :]], axis=0)
        pp1 = jnp.concatenate([pk[1:, :], zrow], axis=0)
        xloc = lax.broadcasted_iota(jnp.int32, (BLK, CH), 0) % NX
        zero = jnp.zeros((BLK, CH), jnp.int32)
        o_ref[:, 0:CH] = jnp.where(xloc != 0, pm1, zero)
        o_ref[:, CH:2 * CH] = pk
        o_ref[:, 2 * CH:] = jnp.where(xloc != NX - 1, pp1, zero)


def _stem(xt, stem_wt, stem_b2):
    nblk = STEM_ROWS // BLK
    cap = NPIX // BLK - 1
    return pl.pallas_call(
        _stem_body,
        grid=(nblk,),
        in_specs=[
            pl.BlockSpec((BLK, C), lambda i: (jnp.minimum(i, cap), 0)),
            pl.BlockSpec((C, CP), lambda i: (0, 0)),
            pl.BlockSpec((1, CP), lambda i: (0, 0)),
        ],
        out_specs=pl.BlockSpec((BLK, 3 * CH), lambda i: (i, 0)),
        out_shape=jax.ShapeDtypeStruct((STEM_ROWS, 3 * CH), jnp.int32),
    )(xt, stem_wt, stem_b2)


_TAPS = [(dy, dx) for dy in (-1, 0, 1) for dx in (-1, 0, 1)]


DEPTH = 2


def _gather_body(stem_hbm, bi_hbm, yi_hbm, xi_hbm, g_hbm, *refs):
    (bv, yv, xv), taps = refs[0:3], refs[3:6]
    bufs = refs[6:6 + DEPTH]
    gsem = refs[6 + DEPTH:6 + 2 * DEPTH]
    wsem = refs[6 + 2 * DEPTH:6 + 3 * DEPTH]
    wid = lax.axis_index("s") * 2 + lax.axis_index("c")
    base = wid * BPW
    pltpu.sync_copy(bi_hbm.at[pl.ds(base, BPW)], bv)
    pltpu.sync_copy(yi_hbm.at[pl.ds(base, BPW)], yv)
    pltpu.sync_copy(xi_hbm.at[pl.ds(base, BPW)], xv)

    for j in range(BPW // 16):
        sl = pl.ds(j * 16, 16)
        b = bv[sl]
        y = yv[sl]
        x = xv[sl]
        flat = (b * NY + y) * NX + x
        taps[0][sl] = jnp.where(y >= 1, flat - NX, NPIX)
        taps[1][sl] = flat
        taps[2][sl] = jnp.where(y <= NY - 2, flat + NX, NPIX)

    chunks = [(t, cc) for t in range(3) for cc in range(NCH)]
    n = len(chunks)

    def _start_gather(i, b):
        t, cc = chunks[i]
        return pltpu.async_copy(
            stem_hbm.at[taps[t].at[pl.ds(cc * GCH, GCH)]], bufs[b], gsem[b])

    def _start_write(i, b):
        t, cc = chunks[i]
        return pltpu.async_copy(
            bufs[b], g_hbm.at[t, pl.ds(base + cc * GCH, GCH)], wsem[b])

    gdesc = [None] * DEPTH
    wdesc = [None] * DEPTH
    for i in range(min(DEPTH, n)):
        gdesc[i] = _start_gather(i, i)
    for i in range(n):
        b = i % DEPTH
        gdesc[b].wait()
        wdesc[b] = _start_write(i, b)
        j = i + DEPTH
        if j < n:
            wdesc[b].wait()
            gdesc[b] = _start_gather(j, b)
            wdesc[b] = None
    for b in range(DEPTH):
        if wdesc[b] is not None:
            wdesc[b].wait()


def _gather(stem_full, bi, yi, xi):
    mesh = plsc.VectorSubcoreMesh(core_axis_name="c", subcore_axis_name="s")
    return pl.kernel(
        _gather_body,
        out_type=jax.ShapeDtypeStruct((3, KH, 3 * CH), jnp.int32),
        mesh=mesh,
        scratch_types=(
            [pltpu.VMEM((BPW,), jnp.int32) for _ in range(3)]
            + [pltpu.VMEM((BPW,), jnp.int32) for _ in range(3)]
            + [pltpu.VMEM((GCH, 3 * CH), jnp.int32) for _ in range(DEPTH)]
            + [pltpu.SemaphoreType.DMA for _ in range(3 * DEPTH)]
        ),
    )(stem_full, bi, yi, xi)


def _head_body(g_ref, wcl_ref, wch_ref, wrl_ref, wrh_ref, cb_ref, rb_ref,
               wro_ref, wcb_ref, hb_ref, o_ref):
    accc = jnp.zeros((KB, C), jnp.float32) + cb_ref[...]
    accr = jnp.zeros((KB, C), jnp.float32) + rb_ref[...]
    for d in range(3):
        gu = lax.bitcast_convert_type(g_ref[d], jnp.uint32)
        glo = lax.bitcast_convert_type(gu << 16, jnp.float32)
        ghi = lax.bitcast_convert_type(gu & jnp.uint32(0xFFFF0000), jnp.float32)
        accc += lax.dot_general(glo, wcl_ref[d], (((1,), (1,)), ((), ())),
                                preferred_element_type=jnp.float32)
        accc += lax.dot_general(ghi, wch_ref[d], (((1,), (1,)), ((), ())),
                                preferred_element_type=jnp.float32)
        accr += lax.dot_general(glo, wrl_ref[d], (((1,), (1,)), ((), ())),
                                preferred_element_type=jnp.float32)
        accr += lax.dot_general(ghi, wrh_ref[d], (((1,), (1,)), ((), ())),
                                preferred_element_type=jnp.float32)
    cls_f = accc * jax.nn.sigmoid(accc)
    reg_f = accr * jax.nn.sigmoid(accr)
    out = lax.dot_general(reg_f, wro_ref[...], (((1,), (1,)), ((), ())),
                          preferred_element_type=jnp.float32)
    out += lax.dot_general(cls_f, wcb_ref[...], (((1,), (1,)), ((), ())),
                           preferred_element_type=jnp.float32)
    o_ref[...] = out + hb_ref[...]


def _heads(g, wcl, wch, wrl, wrh, cb2, rb2, wro, wcb, hbias):
    return pl.pallas_call(
        _head_body,
        grid=(KH // KB,),
        in_specs=[
            pl.BlockSpec((3, KB, 3 * CH), lambda i: (0, i, 0)),
            pl.BlockSpec((3, C, 3 * CH), lambda i: (0, 0, 0)),
            pl.BlockSpec((3, C, 3 * CH), lambda i: (0, 0, 0)),
            pl.BlockSpec((3, C, 3 * CH), lambda i: (0, 0, 0)),
            pl.BlockSpec((3, C, 3 * CH), lambda i: (0, 0, 0)),
            pl.BlockSpec((1, C), lambda i: (0, 0)),
            pl.BlockSpec((1, C), lambda i: (0, 0)),
            pl.BlockSpec((OUT_W, C), lambda i: (0, 0)),
            pl.BlockSpec((OUT_W, C), lambda i: (0, 0)),
            pl.BlockSpec((1, OUT_W), lambda i: (0, 0)),
        ],
        out_specs=pl.BlockSpec((KB, OUT_W), lambda i: (i, 0)),
        out_shape=jax.ShapeDtypeStruct((KH, OUT_W), jnp.float32),
    )(g, wcl, wch, wrl, wrh, cb2, rb2, wro, wcb, hbias)


def kernel(x, indices, stem_w, stem_b, cls_conv_w, cls_conv_b,
           reg_conv_w, reg_conv_b, cls_pred_w, cls_pred_b,
           reg_pred_w, reg_pred_b, obj_pred_w, obj_pred_b):
    xt = jnp.transpose(x.reshape(BS, C, NY * NX), (0, 2, 1)).reshape(NPIX, C)
    stem_wt = jnp.pad(stem_w, ((0, CP - C), (0, 0))).T
    stem_bp = jnp.pad(stem_b, (0, CP - C)).reshape(1, CP)
    stem_full = _stem(xt, stem_wt, stem_bp)

    idx32 = indices.astype(jnp.int32)

    def _tapw(w):
        wp = jnp.pad(jnp.transpose(w.reshape(C, C, 9), (2, 0, 1)),
                     ((0, 0), (0, 0), (0, CP - C)))
        lo = wp[:, :, :CH].reshape(3, 3, C, CH).transpose(0, 2, 1, 3).reshape(
            3, C, 3 * CH)
        hi = wp[:, :, CH:].reshape(3, 3, C, CH).transpose(0, 2, 1, 3).reshape(
            3, C, 3 * CH)
        return lo, hi

    wcl, wch = _tapw(cls_conv_w)
    wrl, wrh = _tapw(reg_conv_w)
    wro = jnp.concatenate(
        [reg_pred_w, obj_pred_w, jnp.zeros((NC, C), jnp.float32)], axis=0)
    wcb = jnp.concatenate(
        [jnp.zeros((4 + 1, C), jnp.float32), cls_pred_w], axis=0)
    hbias = jnp.concatenate(
        [reg_pred_b, obj_pred_b, cls_pred_b]).reshape(1, OUT_W)

    outs = []
    for p in range(SPLIT):
        lo = p * KH
        g = _gather(stem_full, lax.dynamic_slice_in_dim(idx32[:, 0], lo, KH),
                    lax.dynamic_slice_in_dim(idx32[:, 1], lo, KH),
                    lax.dynamic_slice_in_dim(idx32[:, 2], lo, KH))
        outs.append(_heads(g, wcl, wch, wrl, wrh, cls_conv_b.reshape(1, C),
                           reg_conv_b.reshape(1, C), wro, wcb, hbias))
    return jnp.concatenate(outs, axis=0)


# speedup vs baseline: 1.5944x; 1.0034x over previous
import functools

import jax
import jax.numpy as jnp
from jax import lax
from jax.experimental import pallas as pl
from jax.experimental.pallas import tpu as pltpu
from jax.experimental.pallas import tpu_sc as plsc

BS, C, NY, NX = 8, 192, 64, 64
NC, NA = 80, 1
K = 8192
NPIX = BS * NY * NX
BLK = 1024
STEM_ROWS = NPIX + BLK

NWORK = 32
SPLIT = 2
KH = K // SPLIT
BPW = KH // NWORK
GCH = 128
NCH = BPW // GCH

KB = 1024
OUT_W = 4 + 1 + NC
CP = 256
CH = 128


def _stem_body(x_ref, w_ref, b_ref, o_ref):
    i = pl.program_id(0)
    last = pl.num_programs(0) - 1

    @pl.when(i == last)
    def _():
        o_ref[...] = jnp.zeros_like(o_ref)

    @pl.when(i < last)
    def _():
        acc = lax.dot_general(x_ref[...], w_ref[...], (((1,), (0,)), ((), ())),
                              preferred_element_type=jnp.float32)
        acc = acc + b_ref[...]
        act = acc * jax.nn.sigmoid(acc)
        lo = lax.bitcast_convert_type(
            act[:, :CH].astype(jnp.bfloat16), jnp.uint16).astype(jnp.uint32)
        hi = lax.bitcast_convert_type(
            act[:, CH:].astype(jnp.bfloat16), jnp.uint16).astype(jnp.uint32)
        pk = lax.bitcast_convert_type((hi << 16) | lo, jnp.int32)
        zrow = jnp.zeros((1, CH), jnp.int32)
        pm1 = jnp.concatenate([zrow, pk[:BLK - 1, :]], axis=0)
        pp1 = jnp.concatenate([pk[1:, :], zrow], axis=0)
        xloc = lax.broadcasted_iota(jnp.int32, (BLK, CH), 0) % NX
        zero = jnp.zeros((BLK, CH), jnp.int32)
        o_ref[:, 0:CH] = jnp.where(xloc != 0, pm1, zero)
        o_ref[:, CH:2 * CH] = pk
        o_ref[:, 2 * CH:] = jnp.where(xloc != NX - 1, pp1, zero)


def _stem(xt, stem_wt, stem_b2):
    nblk = STEM_ROWS // BLK
    cap = NPIX // BLK - 1
    return pl.pallas_call(
        _stem_body,
        grid=(nblk,),
        in_specs=[
            pl.BlockSpec((BLK, C), lambda i: (jnp.minimum(i, cap), 0)),
            pl.BlockSpec((C, CP), lambda i: (0, 0)),
            pl.BlockSpec((1, CP), lambda i: (0, 0)),
        ],
        out_specs=pl.BlockSpec((BLK, 3 * CH), lambda i: (i, 0)),
        out_shape=jax.ShapeDtypeStruct((STEM_ROWS, 3 * CH), jnp.int32),
    )(xt, stem_wt, stem_b2)


_TAPS = [(dy, dx) for dy in (-1, 0, 1) for dx in (-1, 0, 1)]


DEPTH = 2


def _gather_body(stem_hbm, bi_hbm, yi_hbm, xi_hbm, g_hbm, *refs):
    (bv, yv, xv), taps = refs[0:3], refs[3:6]
    bufs = refs[6:6 + DEPTH]
    gsem = refs[6 + DEPTH:6 + 2 * DEPTH]
    wsem = refs[6 + 2 * DEPTH:6 + 3 * DEPTH]
    wid = lax.axis_index("s") * 2 + lax.axis_index("c")
    base = wid * BPW
    pltpu.sync_copy(bi_hbm.at[pl.ds(base, BPW)], bv)
    pltpu.sync_copy(yi_hbm.at[pl.ds(base, BPW)], yv)
    pltpu.sync_copy(xi_hbm.at[pl.ds(base, BPW)], xv)

    for j in range(BPW // 16):
        sl = pl.ds(j * 16, 16)
        b = bv[sl]
        y = yv[sl]
        x = xv[sl]
        flat = (b * NY + y) * NX + x
        taps[0][sl] = jnp.where(y >= 1, flat - NX, NPIX)
        taps[1][sl] = flat
        taps[2][sl] = jnp.where(y <= NY - 2, flat + NX, NPIX)

    chunks = [(t, cc) for t in range(3) for cc in range(NCH)]
    n = len(chunks)

    def _start_gather(i, b):
        t, cc = chunks[i]
        return pltpu.async_copy(
            stem_hbm.at[taps[t].at[pl.ds(cc * GCH, GCH)]], bufs[b], gsem[b])

    def _start_write(i, b):
        t, cc = chunks[i]
        return pltpu.async_copy(
            bufs[b], g_hbm.at[t, pl.ds(base + cc * GCH, GCH)], wsem[b])

    gdesc = [None] * DEPTH
    wdesc = [None] * DEPTH
    for i in range(min(DEPTH, n)):
        gdesc[i] = _start_gather(i, i)
    for i in range(n):
        b = i % DEPTH
        gdesc[b].wait()
        wdesc[b] = _start_write(i, b)
        j = i + DEPTH
        if j < n:
            wdesc[b].wait()
            gdesc[b] = _start_gather(j, b)
            wdesc[b] = None
    for b in range(DEPTH):
        if wdesc[b] is not None:
            wdesc[b].wait()


def _gather(stem_full, bi, yi, xi):
    mesh = plsc.VectorSubcoreMesh(core_axis_name="c", subcore_axis_name="s")
    return pl.kernel(
        _gather_body,
        out_type=jax.ShapeDtypeStruct((3, KH, 3 * CH), jnp.int32),
        mesh=mesh,
        scratch_types=(
            [pltpu.VMEM((BPW,), jnp.int32) for _ in range(3)]
            + [pltpu.VMEM((BPW,), jnp.int32) for _ in range(3)]
            + [pltpu.VMEM((GCH, 3 * CH), jnp.int32) for _ in range(DEPTH)]
            + [pltpu.SemaphoreType.DMA for _ in range(3 * DEPTH)]
        ),
    )(stem_full, bi, yi, xi)


def _head_body(g_ref, wlo_ref, whi_ref, b2_ref, wro_ref, wcb_ref, hb_ref,
               o_ref):
    acc = jnp.zeros((KB, 2 * CP), jnp.float32) + b2_ref[...]
    for d in range(3):
        gu = lax.bitcast_convert_type(g_ref[d], jnp.uint32)
        glo = lax.bitcast_convert_type(gu << 16, jnp.float32)
        ghi = lax.bitcast_convert_type(gu & jnp.uint32(0xFFFF0000), jnp.float32)
        acc += lax.dot_general(glo, wlo_ref[d], (((1,), (1,)), ((), ())),
                               preferred_element_type=jnp.float32)
        acc += lax.dot_general(ghi, whi_ref[d], (((1,), (1,)), ((), ())),
                               preferred_element_type=jnp.float32)
    feat = acc * jax.nn.sigmoid(acc)
    out = lax.dot_general(feat[:, CP:], wro_ref[...], (((1,), (1,)), ((), ())),
                          preferred_element_type=jnp.float32)
    out += lax.dot_general(feat[:, :CP], wcb_ref[...], (((1,), (1,)), ((), ())),
                           preferred_element_type=jnp.float32)
    o_ref[...] = out + hb_ref[...]


def _heads(g, wlo, whi, b2, wro, wcb, hbias):
    return pl.pallas_call(
        _head_body,
        grid=(KH // KB,),
        in_specs=[
            pl.BlockSpec((3, KB, 3 * CH), lambda i: (0, i, 0)),
            pl.BlockSpec((3, 2 * CP, 3 * CH), lambda i: (0, 0, 0)),
            pl.BlockSpec((3, 2 * CP, 3 * CH), lambda i: (0, 0, 0)),
            pl.BlockSpec((1, 2 * CP), lambda i: (0, 0)),
            pl.BlockSpec((OUT_W, CP), lambda i: (0, 0)),
            pl.BlockSpec((OUT_W, CP), lambda i: (0, 0)),
            pl.BlockSpec((1, OUT_W), lambda i: (0, 0)),
        ],
        out_specs=pl.BlockSpec((KB, OUT_W), lambda i: (i, 0)),
        out_shape=jax.ShapeDtypeStruct((KH, OUT_W), jnp.float32),
    )(g, wlo, whi, b2, wro, wcb, hbias)


def kernel(x, indices, stem_w, stem_b, cls_conv_w, cls_conv_b,
           reg_conv_w, reg_conv_b, cls_pred_w, cls_pred_b,
           reg_pred_w, reg_pred_b, obj_pred_w, obj_pred_b):
    xt = jnp.transpose(x.reshape(BS, C, NY * NX), (0, 2, 1)).reshape(NPIX, C)
    stem_wt = jnp.pad(stem_w, ((0, CP - C), (0, 0))).T
    stem_bp = jnp.pad(stem_b, (0, CP - C)).reshape(1, CP)
    stem_full = _stem(xt, stem_wt, stem_bp)

    idx32 = indices.astype(jnp.int32)

    def _tapw(w):
        wp = jnp.pad(jnp.transpose(w.reshape(C, C, 9), (2, 0, 1)),
                     ((0, 0), (0, 0), (0, CP - C)))
        lo = wp[:, :, :CH].reshape(3, 3, C, CH).transpose(0, 2, 1, 3).reshape(
            3, C, 3 * CH)
        hi = wp[:, :, CH:].reshape(3, 3, C, CH).transpose(0, 2, 1, 3).reshape(
            3, C, 3 * CH)
        return lo, hi

    wcl, wch = _tapw(cls_conv_w)
    wrl, wrh = _tapw(reg_conv_w)
    wlo = jnp.concatenate([jnp.pad(wcl, ((0, 0), (0, CP - C), (0, 0))),
                           jnp.pad(wrl, ((0, 0), (0, CP - C), (0, 0)))], axis=1)
    whi = jnp.concatenate([jnp.pad(wch, ((0, 0), (0, CP - C), (0, 0))),
                           jnp.pad(wrh, ((0, 0), (0, CP - C), (0, 0)))], axis=1)
    b2 = jnp.concatenate([jnp.pad(cls_conv_b, (0, CP - C)),
                          jnp.pad(reg_conv_b, (0, CP - C))]).reshape(1, 2 * CP)
    wro = jnp.pad(jnp.concatenate(
        [reg_pred_w, obj_pred_w, jnp.zeros((NC, C), jnp.float32)], axis=0),
        ((0, 0), (0, CP - C)))
    wcb = jnp.pad(jnp.concatenate(
        [jnp.zeros((4 + 1, C), jnp.float32), cls_pred_w], axis=0),
        ((0, 0), (0, CP - C)))
    hbias = jnp.concatenate(
        [reg_pred_b, obj_pred_b, cls_pred_b]).reshape(1, OUT_W)

    outs = []
    for p in range(SPLIT):
        lo = p * KH
        g = _gather(stem_full, lax.dynamic_slice_in_dim(idx32[:, 0], lo, KH),
                    lax.dynamic_slice_in_dim(idx32[:, 1], lo, KH),
                    lax.dynamic_slice_in_dim(idx32[:, 2], lo, KH))
        outs.append(_heads(g, wlo, whi, b2, wro, wcb, hbias))
    return jnp.concatenate(outs, axis=0)


# speedup vs baseline: 1.6981x; 1.0650x over previous
import functools

import jax
import jax.numpy as jnp
from jax import lax
from jax.experimental import pallas as pl
from jax.experimental.pallas import tpu as pltpu
from jax.experimental.pallas import tpu_sc as plsc

BS, C, NY, NX = 8, 192, 64, 64
NC, NA = 80, 1
K = 8192
NPIX = BS * NY * NX
BLK = 2048
STEM_ROWS = NPIX + BLK

NWORK = 32
SPLIT = 2
KH = K // SPLIT
BPW = KH // NWORK
GCH = 128
NCH = BPW // GCH

KB = 1024
OUT_W = 4 + 1 + NC
CP = 256
CH = 128


def _stem_body(x_ref, w_ref, b_ref, o_ref):
    i = pl.program_id(0)
    last = pl.num_programs(0) - 1

    @pl.when(i == last)
    def _():
        o_ref[...] = jnp.zeros_like(o_ref)

    @pl.when(i < last)
    def _():
        acc = lax.dot_general(x_ref[...], w_ref[...], (((1,), (0,)), ((), ())),
                              preferred_element_type=jnp.float32)
        acc = acc + b_ref[...]
        act = acc * jax.nn.sigmoid(acc)
        lo = lax.bitcast_convert_type(
            act[:, :CH].astype(jnp.bfloat16), jnp.uint16).astype(jnp.uint32)
        hi = lax.bitcast_convert_type(
            act[:, CH:].astype(jnp.bfloat16), jnp.uint16).astype(jnp.uint32)
        pk = lax.bitcast_convert_type((hi << 16) | lo, jnp.int32)
        zrow = jnp.zeros((1, CH), jnp.int32)
        pm1 = jnp.concatenate([zrow, pk[:BLK - 1, :]], axis=0)
        pp1 = jnp.concatenate([pk[1:, :], zrow], axis=0)
        xloc = lax.broadcasted_iota(jnp.int32, (BLK, CH), 0) % NX
        zero = jnp.zeros((BLK, CH), jnp.int32)
        o_ref[:, 0:CH] = jnp.where(xloc != 0, pm1, zero)
        o_ref[:, CH:2 * CH] = pk
        o_ref[:, 2 * CH:] = jnp.where(xloc != NX - 1, pp1, zero)


def _stem(xt, stem_wt, stem_b2):
    nblk = STEM_ROWS // BLK
    cap = NPIX // BLK - 1
    return pl.pallas_call(
        _stem_body,
        grid=(nblk,),
        in_specs=[
            pl.BlockSpec((BLK, C), lambda i: (jnp.minimum(i, cap), 0)),
            pl.BlockSpec((C, CP), lambda i: (0, 0)),
            pl.BlockSpec((1, CP), lambda i: (0, 0)),
        ],
        out_specs=pl.BlockSpec((BLK, 3 * CH), lambda i: (i, 0)),
        out_shape=jax.ShapeDtypeStruct((STEM_ROWS, 3 * CH), jnp.int32),
    )(xt, stem_wt, stem_b2)


_TAPS = [(dy, dx) for dy in (-1, 0, 1) for dx in (-1, 0, 1)]


DEPTH = 2


def _gather_body(stem_hbm, bi_hbm, yi_hbm, xi_hbm, g_hbm, *refs):
    (bv, yv, xv), taps = refs[0:3], refs[3:6]
    bufs = refs[6:6 + DEPTH]
    gsem = refs[6 + DEPTH:6 + 2 * DEPTH]
    wsem = refs[6 + 2 * DEPTH:6 + 3 * DEPTH]
    wid = lax.axis_index("s") * 2 + lax.axis_index("c")
    base = wid * BPW
    pltpu.sync_copy(bi_hbm.at[pl.ds(base, BPW)], bv)
    pltpu.sync_copy(yi_hbm.at[pl.ds(base, BPW)], yv)
    pltpu.sync_copy(xi_hbm.at[pl.ds(base, BPW)], xv)

    for j in range(BPW // 16):
        sl = pl.ds(j * 16, 16)
        b = bv[sl]
        y = yv[sl]
        x = xv[sl]
        flat = (b * NY + y) * NX + x
        taps[0][sl] = jnp.where(y >= 1, flat - NX, NPIX)
        taps[1][sl] = flat
        taps[2][sl] = jnp.where(y <= NY - 2, flat + NX, NPIX)

    chunks = [(t, cc) for t in range(3) for cc in range(NCH)]
    n = len(chunks)

    def _start_gather(i, b):
        t, cc = chunks[i]
        return pltpu.async_copy(
            stem_hbm.at[taps[t].at[pl.ds(cc * GCH, GCH)]], bufs[b], gsem[b])

    def _start_write(i, b):
        t, cc = chunks[i]
        return pltpu.async_copy(
            bufs[b], g_hbm.at[t, pl.ds(base + cc * GCH, GCH)], wsem[b])

    gdesc = [None] * DEPTH
    wdesc = [None] * DEPTH
    for i in range(min(DEPTH, n)):
        gdesc[i] = _start_gather(i, i)
    for i in range(n):
        b = i % DEPTH
        gdesc[b].wait()
        wdesc[b] = _start_write(i, b)
        j = i + DEPTH
        if j < n:
            wdesc[b].wait()
            gdesc[b] = _start_gather(j, b)
            wdesc[b] = None
    for b in range(DEPTH):
        if wdesc[b] is not None:
            wdesc[b].wait()


def _gather(stem_full, bi, yi, xi):
    mesh = plsc.VectorSubcoreMesh(core_axis_name="c", subcore_axis_name="s")
    return pl.kernel(
        _gather_body,
        out_type=jax.ShapeDtypeStruct((3, KH, 3 * CH), jnp.int32),
        mesh=mesh,
        scratch_types=(
            [pltpu.VMEM((BPW,), jnp.int32) for _ in range(3)]
            + [pltpu.VMEM((BPW,), jnp.int32) for _ in range(3)]
            + [pltpu.VMEM((GCH, 3 * CH), jnp.int32) for _ in range(DEPTH)]
            + [pltpu.SemaphoreType.DMA for _ in range(3 * DEPTH)]
        ),
    )(stem_full, bi, yi, xi)


def _head_body(g_ref, wlo_ref, whi_ref, b2_ref, wro_ref, wcb_ref, hb_ref,
               o_ref):
    acc = jnp.zeros((KB, 2 * CP), jnp.float32) + b2_ref[...]
    for d in range(3):
        gu = lax.bitcast_convert_type(g_ref[d], jnp.uint32)
        glo = lax.bitcast_convert_type(gu << 16, jnp.float32)
        ghi = lax.bitcast_convert_type(gu & jnp.uint32(0xFFFF0000), jnp.float32)
        acc += lax.dot_general(glo, wlo_ref[d], (((1,), (1,)), ((), ())),
                               preferred_element_type=jnp.float32)
        acc += lax.dot_general(ghi, whi_ref[d], (((1,), (1,)), ((), ())),
                               preferred_element_type=jnp.float32)
    feat = acc * jax.nn.sigmoid(acc)
    out = lax.dot_general(feat[:, CP:], wro_ref[...], (((1,), (1,)), ((), ())),
                          preferred_element_type=jnp.float32)
    out += lax.dot_general(feat[:, :CP], wcb_ref[...], (((1,), (1,)), ((), ())),
                           preferred_element_type=jnp.float32)
    o_ref[...] = out + hb_ref[...]


def _heads(g, wlo, whi, b2, wro, wcb, hbias):
    return pl.pallas_call(
        _head_body,
        grid=(KH // KB,),
        in_specs=[
            pl.BlockSpec((3, KB, 3 * CH), lambda i: (0, i, 0)),
            pl.BlockSpec((3, 2 * CP, 3 * CH), lambda i: (0, 0, 0)),
            pl.BlockSpec((3, 2 * CP, 3 * CH), lambda i: (0, 0, 0)),
            pl.BlockSpec((1, 2 * CP), lambda i: (0, 0)),
            pl.BlockSpec((OUT_W, CP), lambda i: (0, 0)),
            pl.BlockSpec((OUT_W, CP), lambda i: (0, 0)),
            pl.BlockSpec((1, OUT_W), lambda i: (0, 0)),
        ],
        out_specs=pl.BlockSpec((KB, OUT_W), lambda i: (i, 0)),
        out_shape=jax.ShapeDtypeStruct((KH, OUT_W), jnp.float32),
    )(g, wlo, whi, b2, wro, wcb, hbias)


def kernel(x, indices, stem_w, stem_b, cls_conv_w, cls_conv_b,
           reg_conv_w, reg_conv_b, cls_pred_w, cls_pred_b,
           reg_pred_w, reg_pred_b, obj_pred_w, obj_pred_b):
    xt = jnp.transpose(x.reshape(BS, C, NY * NX), (0, 2, 1)).reshape(NPIX, C)
    stem_wt = jnp.pad(stem_w, ((0, CP - C), (0, 0))).T
    stem_bp = jnp.pad(stem_b, (0, CP - C)).reshape(1, CP)
    stem_full = _stem(xt, stem_wt, stem_bp)

    idx32 = indices.astype(jnp.int32)

    def _tapw(w):
        wp = jnp.pad(jnp.transpose(w.reshape(C, C, 9), (2, 0, 1)),
                     ((0, 0), (0, 0), (0, CP - C)))
        lo = wp[:, :, :CH].reshape(3, 3, C, CH).transpose(0, 2, 1, 3).reshape(
            3, C, 3 * CH)
        hi = wp[:, :, CH:].reshape(3, 3, C, CH).transpose(0, 2, 1, 3).reshape(
            3, C, 3 * CH)
        return lo, hi

    wcl, wch = _tapw(cls_conv_w)
    wrl, wrh = _tapw(reg_conv_w)
    wlo = jnp.concatenate([jnp.pad(wcl, ((0, 0), (0, CP - C), (0, 0))),
                           jnp.pad(wrl, ((0, 0), (0, CP - C), (0, 0)))], axis=1)
    whi = jnp.concatenate([jnp.pad(wch, ((0, 0), (0, CP - C), (0, 0))),
                           jnp.pad(wrh, ((0, 0), (0, CP - C), (0, 0)))], axis=1)
    b2 = jnp.concatenate([jnp.pad(cls_conv_b, (0, CP - C)),
                          jnp.pad(reg_conv_b, (0, CP - C))]).reshape(1, 2 * CP)
    wro = jnp.pad(jnp.concatenate(
        [reg_pred_w, obj_pred_w, jnp.zeros((NC, C), jnp.float32)], axis=0),
        ((0, 0), (0, CP - C)))
    wcb = jnp.pad(jnp.concatenate(
        [jnp.zeros((4 + 1, C), jnp.float32), cls_pred_w], axis=0),
        ((0, 0), (0, CP - C)))
    hbias = jnp.concatenate(
        [reg_pred_b, obj_pred_b, cls_pred_b]).reshape(1, OUT_W)

    outs = []
    for p in range(SPLIT):
        lo = p * KH
        g = _gather(stem_full, lax.dynamic_slice_in_dim(idx32[:, 0], lo, KH),
                    lax.dynamic_slice_in_dim(idx32[:, 1], lo, KH),
                    lax.dynamic_slice_in_dim(idx32[:, 2], lo, KH))
        outs.append(_heads(g, wlo, whi, b2, wro, wcb, hbias))
    return jnp.concatenate(outs, axis=0)


# speedup vs baseline: 1.7363x; 1.0225x over previous
import functools

import jax
import jax.numpy as jnp
from jax import lax
from jax.experimental import pallas as pl
from jax.experimental.pallas import tpu as pltpu
from jax.experimental.pallas import tpu_sc as plsc

BS, C, NY, NX = 8, 192, 64, 64
NC, NA = 80, 1
K = 8192
NPIX = BS * NY * NX
BLK = 4096
STEM_ROWS = NPIX + BLK

NWORK = 32
SPLIT = 2
KH = K // SPLIT
BPW = KH // NWORK
GCH = 128
NCH = BPW // GCH

KB = 1024
OUT_W = 4 + 1 + NC
CP = 256
CH = 128


def _stem_body(x_ref, w_ref, b_ref, o_ref):
    i = pl.program_id(0)
    last = pl.num_programs(0) - 1

    @pl.when(i == last)
    def _():
        o_ref[...] = jnp.zeros_like(o_ref)

    @pl.when(i < last)
    def _():
        acc = lax.dot_general(x_ref[...], w_ref[...], (((1,), (0,)), ((), ())),
                              preferred_element_type=jnp.float32)
        acc = acc + b_ref[...]
        act = acc * jax.nn.sigmoid(acc)
        lo = lax.bitcast_convert_type(
            act[:, :CH].astype(jnp.bfloat16), jnp.uint16).astype(jnp.uint32)
        hi = lax.bitcast_convert_type(
            act[:, CH:].astype(jnp.bfloat16), jnp.uint16).astype(jnp.uint32)
        pk = lax.bitcast_convert_type((hi << 16) | lo, jnp.int32)
        zrow = jnp.zeros((1, CH), jnp.int32)
        pm1 = jnp.concatenate([zrow, pk[:BLK - 1, :]], axis=0)
        pp1 = jnp.concatenate([pk[1:, :], zrow], axis=0)
        xloc = lax.broadcasted_iota(jnp.int32, (BLK, CH), 0) % NX
        zero = jnp.zeros((BLK, CH), jnp.int32)
        o_ref[:, 0:CH] = jnp.where(xloc != 0, pm1, zero)
        o_ref[:, CH:2 * CH] = pk
        o_ref[:, 2 * CH:] = jnp.where(xloc != NX - 1, pp1, zero)


def _stem(xt, stem_wt, stem_b2):
    nblk = STEM_ROWS // BLK
    cap = NPIX // BLK - 1
    return pl.pallas_call(
        _stem_body,
        grid=(nblk,),
        in_specs=[
            pl.BlockSpec((BLK, C), lambda i: (jnp.minimum(i, cap), 0)),
            pl.BlockSpec((C, CP), lambda i: (0, 0)),
            pl.BlockSpec((1, CP), lambda i: (0, 0)),
        ],
        out_specs=pl.BlockSpec((BLK, 3 * CH), lambda i: (i, 0)),
        out_shape=jax.ShapeDtypeStruct((STEM_ROWS, 3 * CH), jnp.int32),
    )(xt, stem_wt, stem_b2)


_TAPS = [(dy, dx) for dy in (-1, 0, 1) for dx in (-1, 0, 1)]


DEPTH = 2


def _gather_body(stem_hbm, bi_hbm, yi_hbm, xi_hbm, g_hbm, *refs):
    (bv, yv, xv), taps = refs[0:3], refs[3:6]
    bufs = refs[6:6 + DEPTH]
    gsem = refs[6 + DEPTH:6 + 2 * DEPTH]
    wsem = refs[6 + 2 * DEPTH:6 + 3 * DEPTH]
    wid = lax.axis_index("s") * 2 + lax.axis_index("c")
    base = wid * BPW
    pltpu.sync_copy(bi_hbm.at[pl.ds(base, BPW)], bv)
    pltpu.sync_copy(yi_hbm.at[pl.ds(base, BPW)], yv)
    pltpu.sync_copy(xi_hbm.at[pl.ds(base, BPW)], xv)

    for j in range(BPW // 16):
        sl = pl.ds(j * 16, 16)
        b = bv[sl]
        y = yv[sl]
        x = xv[sl]
        flat = (b * NY + y) * NX + x
        taps[0][sl] = jnp.where(y >= 1, flat - NX, NPIX)
        taps[1][sl] = flat
        taps[2][sl] = jnp.where(y <= NY - 2, flat + NX, NPIX)

    chunks = [(t, cc) for t in range(3) for cc in range(NCH)]
    n = len(chunks)

    def _start_gather(i, b):
        t, cc = chunks[i]
        return pltpu.async_copy(
            stem_hbm.at[taps[t].at[pl.ds(cc * GCH, GCH)]], bufs[b], gsem[b])

    def _start_write(i, b):
        t, cc = chunks[i]
        return pltpu.async_copy(
            bufs[b], g_hbm.at[t, pl.ds(base + cc * GCH, GCH)], wsem[b])

    gdesc = [None] * DEPTH
    wdesc = [None] * DEPTH
    for i in range(min(DEPTH, n)):
        gdesc[i] = _start_gather(i, i)
    for i in range(n):
        b = i % DEPTH
        gdesc[b].wait()
        wdesc[b] = _start_write(i, b)
        j = i + DEPTH
        if j < n:
            wdesc[b].wait()
            gdesc[b] = _start_gather(j, b)
            wdesc[b] = None
    for b in range(DEPTH):
        if wdesc[b] is not None:
            wdesc[b].wait()


def _gather(stem_full, bi, yi, xi):
    mesh = plsc.VectorSubcoreMesh(core_axis_name="c", subcore_axis_name="s")
    return pl.kernel(
        _gather_body,
        out_type=jax.ShapeDtypeStruct((3, KH, 3 * CH), jnp.int32),
        mesh=mesh,
        scratch_types=(
            [pltpu.VMEM((BPW,), jnp.int32) for _ in range(3)]
            + [pltpu.VMEM((BPW,), jnp.int32) for _ in range(3)]
            + [pltpu.VMEM((GCH, 3 * CH), jnp.int32) for _ in range(DEPTH)]
            + [pltpu.SemaphoreType.DMA for _ in range(3 * DEPTH)]
        ),
    )(stem_full, bi, yi, xi)


def _head_body(g_ref, wlo_ref, whi_ref, b2_ref, wro_ref, wcb_ref, hb_ref,
               o_ref):
    acc = jnp.zeros((KB, 2 * CP), jnp.float32) + b2_ref[...]
    for d in range(3):
        gu = lax.bitcast_convert_type(g_ref[d], jnp.uint32)
        glo = lax.bitcast_convert_type(gu << 16, jnp.float32)
        ghi = lax.bitcast_convert_type(gu & jnp.uint32(0xFFFF0000), jnp.float32)
        acc += lax.dot_general(glo, wlo_ref[d], (((1,), (1,)), ((), ())),
                               preferred_element_type=jnp.float32)
        acc += lax.dot_general(ghi, whi_ref[d], (((1,), (1,)), ((), ())),
                               preferred_element_type=jnp.float32)
    feat = acc * jax.nn.sigmoid(acc)
    out = lax.dot_general(feat[:, CP:], wro_ref[...], (((1,), (1,)), ((), ())),
                          preferred_element_type=jnp.float32)
    out += lax.dot_general(feat[:, :CP], wcb_ref[...], (((1,), (1,)), ((), ())),
                           preferred_element_type=jnp.float32)
    o_ref[...] = out + hb_ref[...]


def _heads(g, wlo, whi, b2, wro, wcb, hbias):
    return pl.pallas_call(
        _head_body,
        grid=(KH // KB,),
        in_specs=[
            pl.BlockSpec((3, KB, 3 * CH), lambda i: (0, i, 0)),
            pl.BlockSpec((3, 2 * CP, 3 * CH), lambda i: (0, 0, 0)),
            pl.BlockSpec((3, 2 * CP, 3 * CH), lambda i: (0, 0, 0)),
            pl.BlockSpec((1, 2 * CP), lambda i: (0, 0)),
            pl.BlockSpec((OUT_W, CP), lambda i: (0, 0)),
            pl.BlockSpec((OUT_W, CP), lambda i: (0, 0)),
            pl.BlockSpec((1, OUT_W), lambda i: (0, 0)),
        ],
        out_specs=pl.BlockSpec((KB, OUT_W), lambda i: (i, 0)),
        out_shape=jax.ShapeDtypeStruct((KH, OUT_W), jnp.float32),
    )(g, wlo, whi, b2, wro, wcb, hbias)


def kernel(x, indices, stem_w, stem_b, cls_conv_w, cls_conv_b,
           reg_conv_w, reg_conv_b, cls_pred_w, cls_pred_b,
           reg_pred_w, reg_pred_b, obj_pred_w, obj_pred_b):
    xt = jnp.transpose(x.reshape(BS, C, NY * NX), (0, 2, 1)).reshape(NPIX, C)
    stem_wt = jnp.pad(stem_w, ((0, CP - C), (0, 0))).T
    stem_bp = jnp.pad(stem_b, (0, CP - C)).reshape(1, CP)
    stem_full = _stem(xt, stem_wt, stem_bp)

    idx32 = indices.astype(jnp.int32)

    def _tapw(w):
        wp = jnp.pad(jnp.transpose(w.reshape(C, C, 9), (2, 0, 1)),
                     ((0, 0), (0, 0), (0, CP - C)))
        lo = wp[:, :, :CH].reshape(3, 3, C, CH).transpose(0, 2, 1, 3).reshape(
            3, C, 3 * CH)
        hi = wp[:, :, CH:].reshape(3, 3, C, CH).transpose(0, 2, 1, 3).reshape(
            3, C, 3 * CH)
        return lo, hi

    wcl, wch = _tapw(cls_conv_w)
    wrl, wrh = _tapw(reg_conv_w)
    wlo = jnp.concatenate([jnp.pad(wcl, ((0, 0), (0, CP - C), (0, 0))),
                           jnp.pad(wrl, ((0, 0), (0, CP - C), (0, 0)))], axis=1)
    whi = jnp.concatenate([jnp.pad(wch, ((0, 0), (0, CP - C), (0, 0))),
                           jnp.pad(wrh, ((0, 0), (0, CP - C), (0, 0)))], axis=1)
    b2 = jnp.concatenate([jnp.pad(cls_conv_b, (0, CP - C)),
                          jnp.pad(reg_conv_b, (0, CP - C))]).reshape(1, 2 * CP)
    wro = jnp.pad(jnp.concatenate(
        [reg_pred_w, obj_pred_w, jnp.zeros((NC, C), jnp.float32)], axis=0),
        ((0, 0), (0, CP - C)))
    wcb = jnp.pad(jnp.concatenate(
        [jnp.zeros((4 + 1, C), jnp.float32), cls_pred_w], axis=0),
        ((0, 0), (0, CP - C)))
    hbias = jnp.concatenate(
        [reg_pred_b, obj_pred_b, cls_pred_b]).reshape(1, OUT_W)

    outs = []
    for p in range(SPLIT):
        lo = p * KH
        g = _gather(stem_full, lax.dynamic_slice_in_dim(idx32[:, 0], lo, KH),
                    lax.dynamic_slice_in_dim(idx32[:, 1], lo, KH),
                    lax.dynamic_slice_in_dim(idx32[:, 2], lo, KH))
        outs.append(_heads(g, wlo, whi, b2, wro, wcb, hbias))
    return jnp.concatenate(outs, axis=0)


# speedup vs baseline: 1.7367x; 1.0002x over previous
import jax
import jax.numpy as jnp
from jax import lax
from jax.experimental import pallas as pl
from jax.experimental.pallas import tpu as pltpu
from jax.experimental.pallas import tpu_sc as plsc

BS, C, NY, NX = 8, 192, 64, 64
NC, NA = 80, 1
K = 8192
NPIX = BS * NY * NX
BLK = 4096
STEM_ROWS = NPIX + BLK

NWORK = 32
SPLIT = 2
KH = K // SPLIT
BPW = KH // NWORK
GCH = 128
NCH = BPW // GCH

KB = 1024
OUT_W = 4 + 1 + NC
CP = 256
CH = 128


def _stem_body(x_ref, w_ref, b_ref, o_ref):
    i = pl.program_id(0)
    last = pl.num_programs(0) - 1

    @pl.when(i == last)
    def _():
        o_ref[...] = jnp.zeros_like(o_ref)

    @pl.when(i < last)
    def _():
        acc = lax.dot_general(x_ref[...], w_ref[...], (((1,), (0,)), ((), ())),
                              preferred_element_type=jnp.float32)
        acc = acc + b_ref[...]
        act = acc * jax.nn.sigmoid(acc)
        lo = lax.bitcast_convert_type(
            act[:, :CH].astype(jnp.bfloat16), jnp.uint16).astype(jnp.uint32)
        hi = lax.bitcast_convert_type(
            act[:, CH:].astype(jnp.bfloat16), jnp.uint16).astype(jnp.uint32)
        pk = lax.bitcast_convert_type((hi << 16) | lo, jnp.int32)
        zrow = jnp.zeros((1, CH), jnp.int32)
        pm1 = jnp.concatenate([zrow, pk[:BLK - 1, :]], axis=0)
        pp1 = jnp.concatenate([pk[1:, :], zrow], axis=0)
        xloc = lax.broadcasted_iota(jnp.int32, (BLK, CH), 0) % NX
        zero = jnp.zeros((BLK, CH), jnp.int32)
        o_ref[:, 0:CH] = jnp.where(xloc != 0, pm1, zero)
        o_ref[:, CH:2 * CH] = pk
        o_ref[:, 2 * CH:] = jnp.where(xloc != NX - 1, pp1, zero)


def _stem(xt, stem_wt, stem_b2):
    nblk = STEM_ROWS // BLK
    cap = NPIX // BLK - 1
    return pl.pallas_call(
        _stem_body,
        grid=(nblk,),
        in_specs=[
            pl.BlockSpec((BLK, C), lambda i: (jnp.minimum(i, cap), 0)),
            pl.BlockSpec((C, CP), lambda i: (0, 0)),
            pl.BlockSpec((1, CP), lambda i: (0, 0)),
        ],
        out_specs=pl.BlockSpec((BLK, 3 * CH), lambda i: (i, 0)),
        out_shape=jax.ShapeDtypeStruct((STEM_ROWS, 3 * CH), jnp.int32),
    )(xt, stem_wt, stem_b2)


DEPTH = 2


def _gather_body(stem_hbm, bi_hbm, yi_hbm, xi_hbm, g_hbm, *refs):
    (bv, yv, xv), taps = refs[0:3], refs[3:6]
    bufs = refs[6:6 + DEPTH]
    gsem = refs[6 + DEPTH:6 + 2 * DEPTH]
    wsem = refs[6 + 2 * DEPTH:6 + 3 * DEPTH]
    wid = lax.axis_index("s") * 2 + lax.axis_index("c")
    base = wid * BPW
    pltpu.sync_copy(bi_hbm.at[pl.ds(base, BPW)], bv)
    pltpu.sync_copy(yi_hbm.at[pl.ds(base, BPW)], yv)
    pltpu.sync_copy(xi_hbm.at[pl.ds(base, BPW)], xv)

    for j in range(BPW // 16):
        sl = pl.ds(j * 16, 16)
        b = bv[sl]
        y = yv[sl]
        x = xv[sl]
        flat = (b * NY + y) * NX + x
        taps[0][sl] = jnp.where(y >= 1, flat - NX, NPIX)
        taps[1][sl] = flat
        taps[2][sl] = jnp.where(y <= NY - 2, flat + NX, NPIX)

    chunks = [(t, cc) for t in range(3) for cc in range(NCH)]
    n = len(chunks)

    def _start_gather(i, b):
        t, cc = chunks[i]
        return pltpu.async_copy(
            stem_hbm.at[taps[t].at[pl.ds(cc * GCH, GCH)]], bufs[b], gsem[b])

    def _start_write(i, b):
        t, cc = chunks[i]
        return pltpu.async_copy(
            bufs[b], g_hbm.at[t, pl.ds(base + cc * GCH, GCH)], wsem[b])

    gdesc = [None] * DEPTH
    wdesc = [None] * DEPTH
    for i in range(min(DEPTH, n)):
        gdesc[i] = _start_gather(i, i)
    for i in range(n):
        b = i % DEPTH
        gdesc[b].wait()
        wdesc[b] = _start_write(i, b)
        j = i + DEPTH
        if j < n:
            wdesc[b].wait()
            gdesc[b] = _start_gather(j, b)
            wdesc[b] = None
    for b in range(DEPTH):
        if wdesc[b] is not None:
            wdesc[b].wait()


def _gather(stem_full, bi, yi, xi):
    mesh = plsc.VectorSubcoreMesh(core_axis_name="c", subcore_axis_name="s")
    return pl.kernel(
        _gather_body,
        out_type=jax.ShapeDtypeStruct((3, KH, 3 * CH), jnp.int32),
        mesh=mesh,
        scratch_types=(
            [pltpu.VMEM((BPW,), jnp.int32) for _ in range(3)]
            + [pltpu.VMEM((BPW,), jnp.int32) for _ in range(3)]
            + [pltpu.VMEM((GCH, 3 * CH), jnp.int32) for _ in range(DEPTH)]
            + [pltpu.SemaphoreType.DMA for _ in range(3 * DEPTH)]
        ),
    )(stem_full, bi, yi, xi)


def _head_body(g_ref, wlo_ref, whi_ref, b2_ref, wro_ref, wcb_ref, hb_ref,
               o_ref):
    acc = jnp.zeros((KB, 2 * CP), jnp.float32) + b2_ref[...]
    for d in range(3):
        gu = lax.bitcast_convert_type(g_ref[d], jnp.uint32)
        glo = lax.bitcast_convert_type(gu << 16, jnp.float32)
        ghi = lax.bitcast_convert_type(gu & jnp.uint32(0xFFFF0000), jnp.float32)
        acc += lax.dot_general(glo, wlo_ref[d], (((1,), (1,)), ((), ())),
                               preferred_element_type=jnp.float32)
        acc += lax.dot_general(ghi, whi_ref[d], (((1,), (1,)), ((), ())),
                               preferred_element_type=jnp.float32)
    feat = acc * jax.nn.sigmoid(acc)
    out = lax.dot_general(feat[:, CP:], wro_ref[...], (((1,), (1,)), ((), ())),
                          preferred_element_type=jnp.float32)
    out += lax.dot_general(feat[:, :CP], wcb_ref[...], (((1,), (1,)), ((), ())),
                           preferred_element_type=jnp.float32)
    o_ref[...] = out + hb_ref[...]


def _heads(g, wlo, whi, b2, wro, wcb, hbias):
    return pl.pallas_call(
        _head_body,
        grid=(KH // KB,),
        in_specs=[
            pl.BlockSpec((3, KB, 3 * CH), lambda i: (0, i, 0)),
            pl.BlockSpec((3, 2 * CP, 3 * CH), lambda i: (0, 0, 0)),
            pl.BlockSpec((3, 2 * CP, 3 * CH), lambda i: (0, 0, 0)),
            pl.BlockSpec((1, 2 * CP), lambda i: (0, 0)),
            pl.BlockSpec((OUT_W, CP), lambda i: (0, 0)),
            pl.BlockSpec((OUT_W, CP), lambda i: (0, 0)),
            pl.BlockSpec((1, OUT_W), lambda i: (0, 0)),
        ],
        out_specs=pl.BlockSpec((KB, OUT_W), lambda i: (i, 0)),
        out_shape=jax.ShapeDtypeStruct((KH, OUT_W), jnp.float32),
    )(g, wlo, whi, b2, wro, wcb, hbias)


def kernel(x, indices, stem_w, stem_b, cls_conv_w, cls_conv_b,
           reg_conv_w, reg_conv_b, cls_pred_w, cls_pred_b,
           reg_pred_w, reg_pred_b, obj_pred_w, obj_pred_b):
    xt = jnp.transpose(x.reshape(BS, C, NY * NX), (0, 2, 1)).reshape(NPIX, C)
    stem_wt = jnp.pad(stem_w, ((0, CP - C), (0, 0))).T
    stem_bp = jnp.pad(stem_b, (0, CP - C)).reshape(1, CP)
    stem_full = _stem(xt, stem_wt, stem_bp)

    idx32 = indices.astype(jnp.int32)

    def _tapw(w):
        wp = jnp.pad(jnp.transpose(w.reshape(C, C, 9), (2, 0, 1)),
                     ((0, 0), (0, 0), (0, CP - C)))
        lo = wp[:, :, :CH].reshape(3, 3, C, CH).transpose(0, 2, 1, 3).reshape(
            3, C, 3 * CH)
        hi = wp[:, :, CH:].reshape(3, 3, C, CH).transpose(0, 2, 1, 3).reshape(
            3, C, 3 * CH)
        return lo, hi

    wcl, wch = _tapw(cls_conv_w)
    wrl, wrh = _tapw(reg_conv_w)
    wlo = jnp.concatenate([jnp.pad(wcl, ((0, 0), (0, CP - C), (0, 0))),
                           jnp.pad(wrl, ((0, 0), (0, CP - C), (0, 0)))], axis=1)
    whi = jnp.concatenate([jnp.pad(wch, ((0, 0), (0, CP - C), (0, 0))),
                           jnp.pad(wrh, ((0, 0), (0, CP - C), (0, 0)))], axis=1)
    b2 = jnp.concatenate([jnp.pad(cls_conv_b, (0, CP - C)),
                          jnp.pad(reg_conv_b, (0, CP - C))]).reshape(1, 2 * CP)
    wro = jnp.pad(jnp.concatenate(
        [reg_pred_w, obj_pred_w, jnp.zeros((NC, C), jnp.float32)], axis=0),
        ((0, 0), (0, CP - C)))
    wcb = jnp.pad(jnp.concatenate(
        [jnp.zeros((4 + 1, C), jnp.float32), cls_pred_w], axis=0),
        ((0, 0), (0, CP - C)))
    hbias = jnp.concatenate(
        [reg_pred_b, obj_pred_b, cls_pred_b]).reshape(1, OUT_W)

    outs = []
    for p in range(SPLIT):
        lo = p * KH
        g = _gather(stem_full, lax.dynamic_slice_in_dim(idx32[:, 0], lo, KH),
                    lax.dynamic_slice_in_dim(idx32[:, 1], lo, KH),
                    lax.dynamic_slice_in_dim(idx32[:, 2], lo, KH))
        outs.append(_heads(g, wlo, whi, b2, wro, wcb, hbias))
    return jnp.concatenate(outs, axis=0)
